# Initial kernel scaffold; baseline (speedup 1.0000x reference)
import jax
import jax.numpy as jnp
from jax.experimental import pallas as pl


def kernel(x, batch, W, b):
    raise NotImplementedError("write your pallas kernel here")


# speedup vs baseline: 1.6778x; 1.6778x over previous
import numpy as np

import jax
import jax.numpy as jnp
from jax import lax
from jax.experimental import pallas as pl
from jax.experimental.pallas import tpu as pltpu
from jax.experimental.pallas import tpu_sc as plsc

N = 100000
D = 128
NUM_SEGMENTS = 256
NC = 2
NS = 16
NW = NC * NS
SEG_PER_W = NUM_SEGMENTS // NW
BLK = 256
L = 16


_GDN = lax.GatherDimensionNumbers(
    offset_dims=(), collapsed_slice_dims=(0,), start_index_map=(0,))


def _dg(v, idx):
    return lax.gather(v, idx.reshape(L, 1), _GDN, (1,),
                      mode=lax.GatherScatterMode.PROMISE_IN_BOUNDS)


def _lane_allsum(v):
    iota = lax.iota(jnp.int32, L)
    for sh in (8, 4, 2, 1):
        v = v + _dg(v, (iota + sh) & (L - 1))
    return v


def _sc_body(x_hbm, batch_hbm, w_hbm, b_hbm, offs_hbm, out_hbm,
             w_v, b_v, offs_v, xb_v, idx_v, acc_v):
    wid = lax.axis_index("s") * NC + lax.axis_index("c")

    pltpu.sync_copy(w_hbm, w_v)
    pltpu.sync_copy(b_hbm, b_v)
    pltpu.sync_copy(offs_hbm, offs_v)

    zv = jnp.zeros((L,), jnp.float32)
    for s in range(SEG_PER_W):
        for k in range(D // L):
            acc_v[s, pl.ds(k * L, L)] = zv

    ws = [w_v[pl.ds(k * L, L)] for k in range(D // L)]
    b_bc = _dg(b_v[pl.ds(0, L)], lax.iota(jnp.int32, L) * 0)

    seg_lo = wid * SEG_PER_W
    ovec = offs_v[pl.ds(seg_lo, L)]
    start = ovec[0]
    end = ovec[SEG_PER_W]
    first = (start // 8) * 8
    nblk = lax.max((end - first + BLK - 1) // BLK, 0)

    def blk_body(i, carry):
        p = first + i * BLK
        bs = jnp.minimum(p, N - BLK)
        pltpu.sync_copy(x_hbm.at[pl.ds(bs, BLK)], xb_v)
        pltpu.sync_copy(batch_hbm.at[pl.ds(bs, BLK)], idx_v)
        lo = lax.max(p, start)

        def grp_body(q, c):
            segvec = idx_v[pl.ds(q * L, L)] - seg_lo
            for j in range(L):
                r = q * L + j
                g = bs + r

                @pl.when((g >= lo) & (g < end))
                def _(r=r, seg=segvec[j]):
                    xs = [xb_v[r, pl.ds(k * L, L)] for k in range(D // L)]
                    m = [xs[k] * ws[k] for k in range(D // L)]
                    s01 = (m[0] + m[1]) + (m[2] + m[3])
                    s23 = (m[4] + m[5]) + (m[6] + m[7])
                    zv = _lane_allsum(s01 + s23) + b_bc
                    wt = 1.0 / (1.0 + jnp.exp(-zv))
                    for k in range(D // L):
                        plsc.addupdate(acc_v.at[seg, pl.ds(k * L, L)], xs[k] * wt)

            return c

        lax.fori_loop(0, BLK // L, grp_body, 0, unroll=False)
        return carry

    lax.fori_loop(0, nblk, blk_body, 0, unroll=False)

    pltpu.sync_copy(acc_v, out_hbm.at[pl.ds(seg_lo, SEG_PER_W)])


@jax.jit
def kernel(x, batch, W, b):
    batch_i = batch.astype(jnp.int32)
    offs = jnp.searchsorted(batch_i, jnp.arange(NUM_SEGMENTS + 1, dtype=jnp.int32)
                            ).astype(jnp.int32)
    offs = jnp.pad(offs, (0, 7), constant_values=N)
    w_flat = W.reshape(D).astype(jnp.float32)
    b_pad = jnp.pad(b.astype(jnp.float32), (0, L - 1))

    mesh = plsc.VectorSubcoreMesh(core_axis_name="c", subcore_axis_name="s",
                                  num_cores=NC, num_subcores=NS)
    run = pl.kernel(
        _sc_body,
        out_type=jax.ShapeDtypeStruct((NUM_SEGMENTS, D), jnp.float32),
        mesh=mesh,
        scratch_types=[
            pltpu.VMEM((D,), jnp.float32),
            pltpu.VMEM((L,), jnp.float32),
            pltpu.VMEM((NUM_SEGMENTS + 8,), jnp.int32),
            pltpu.VMEM((BLK, D), jnp.float32),
            pltpu.VMEM((BLK,), jnp.int32),
            pltpu.VMEM((SEG_PER_W, D), jnp.float32),
        ],
    )
    return run(x, batch_i, w_flat, b_pad, offs)


# speedup vs baseline: 1.6955x; 1.0106x over previous
import numpy as np

import jax
import jax.numpy as jnp
from jax import lax
from jax.experimental import pallas as pl
from jax.experimental.pallas import tpu as pltpu
from jax.experimental.pallas import tpu_sc as plsc

N = 100000
D = 128
NUM_SEGMENTS = 256
NC = 2
NS = 16
NW = NC * NS
SEG_PER_W = NUM_SEGMENTS // NW
BLK = 256
L = 16


_GDN = lax.GatherDimensionNumbers(
    offset_dims=(), collapsed_slice_dims=(0,), start_index_map=(0,))


def _dg(v, idx):
    return lax.gather(v, idx.reshape(L, 1), _GDN, (1,),
                      mode=lax.GatherScatterMode.PROMISE_IN_BOUNDS)


def _lane_allsum(v):
    iota = lax.iota(jnp.int32, L)
    for sh in (8, 4, 2, 1):
        v = v + _dg(v, (iota + sh) & (L - 1))
    return v


def _sc_body(x_hbm, batch_hbm, w_hbm, b_hbm, offs_hbm, out_hbm,
             w_v, b_v, offs_v, xb_v, idx_v, acc_v):
    wid = lax.axis_index("s") * NC + lax.axis_index("c")

    pltpu.sync_copy(w_hbm, w_v)
    pltpu.sync_copy(b_hbm, b_v)
    pltpu.sync_copy(offs_hbm, offs_v)

    zv = jnp.zeros((L,), jnp.float32)
    for s in range(SEG_PER_W):
        for k in range(D // L):
            acc_v[s, pl.ds(k * L, L)] = zv

    ws = [w_v[pl.ds(k * L, L)] for k in range(D // L)]
    b_bc = _dg(b_v[pl.ds(0, L)], lax.iota(jnp.int32, L) * 0)

    seg_lo = wid * SEG_PER_W
    ovec = offs_v[pl.ds(seg_lo, L)]
    start = ovec[0]
    end = ovec[SEG_PER_W]
    first = (start // 8) * 8
    nblk = lax.max((end - first + BLK - 1) // BLK, 0)

    def blk_body(i, carry):
        p = first + i * BLK
        bs = jnp.minimum(p, N - BLK)
        pltpu.sync_copy(x_hbm.at[pl.ds(bs, BLK)], xb_v)
        pltpu.sync_copy(batch_hbm.at[pl.ds(bs, BLK)], idx_v)
        lo = lax.max(p, start)

        def grp_body(q, c):
            segvec = idx_v[pl.ds(q * L, L)]
            gvec = (bs + q * L) + lax.iota(jnp.int32, L)
            pen = jnp.where((gvec >= lo) & (gvec < end), 0.0, -1e30)
            for j in range(L):
                r = q * L + j
                seg = (segvec[j] - seg_lo) & (SEG_PER_W - 1)
                xs = [xb_v[r, pl.ds(k * L, L)] for k in range(D // L)]
                m = [xs[k] * ws[k] for k in range(D // L)]
                s01 = (m[0] + m[1]) + (m[2] + m[3])
                s23 = (m[4] + m[5]) + (m[6] + m[7])
                zv = _lane_allsum(s01 + s23) + b_bc + pen[j]
                wt = 1.0 / (1.0 + jnp.exp(-zv))
                for k in range(D // L):
                    plsc.addupdate(acc_v.at[seg, pl.ds(k * L, L)], xs[k] * wt)

            return c

        lax.fori_loop(0, BLK // L, grp_body, 0, unroll=False)
        return carry

    lax.fori_loop(0, nblk, blk_body, 0, unroll=False)

    pltpu.sync_copy(acc_v, out_hbm.at[pl.ds(seg_lo, SEG_PER_W)])


@jax.jit
def kernel(x, batch, W, b):
    batch_i = batch.astype(jnp.int32)
    offs = jnp.searchsorted(batch_i, jnp.arange(NUM_SEGMENTS + 1, dtype=jnp.int32)
                            ).astype(jnp.int32)
    offs = jnp.pad(offs, (0, 7), constant_values=N)
    w_flat = W.reshape(D).astype(jnp.float32)
    b_pad = jnp.pad(b.astype(jnp.float32), (0, L - 1))

    mesh = plsc.VectorSubcoreMesh(core_axis_name="c", subcore_axis_name="s",
                                  num_cores=NC, num_subcores=NS)
    run = pl.kernel(
        _sc_body,
        out_type=jax.ShapeDtypeStruct((NUM_SEGMENTS, D), jnp.float32),
        mesh=mesh,
        scratch_types=[
            pltpu.VMEM((D,), jnp.float32),
            pltpu.VMEM((L,), jnp.float32),
            pltpu.VMEM((NUM_SEGMENTS + 8,), jnp.int32),
            pltpu.VMEM((BLK, D), jnp.float32),
            pltpu.VMEM((BLK,), jnp.int32),
            pltpu.VMEM((SEG_PER_W, D), jnp.float32),
        ],
    )
    return run(x, batch_i, w_flat, b_pad, offs)


# speedup vs baseline: 2.7404x; 1.6163x over previous
import numpy as np

import jax
import jax.numpy as jnp
from jax import lax
from jax.experimental import pallas as pl
from jax.experimental.pallas import tpu as pltpu
from jax.experimental.pallas import tpu_sc as plsc

N = 100000
D = 128
NUM_SEGMENTS = 256
NC = 2
NS = 16
NW = NC * NS
SEG_PER_W = NUM_SEGMENTS // NW
BLK = 256
L = 16


_GDN = lax.GatherDimensionNumbers(
    offset_dims=(), collapsed_slice_dims=(0,), start_index_map=(0,))


def _dg(v, idx):
    return lax.gather(v, idx.reshape(L, 1), _GDN, (1,),
                      mode=lax.GatherScatterMode.PROMISE_IN_BOUNDS)


def _lane_allsum(v):
    iota = lax.iota(jnp.int32, L)
    for sh in (8, 4, 2, 1):
        v = v + _dg(v, (iota + sh) & (L - 1))
    return v


def _sc_body(x_hbm, batch_hbm, w_hbm, b_hbm, offs_hbm, out_hbm,
             w_v, b_v, offs_v, xb_v, idx_v, acc_v):
    wid = lax.axis_index("s") * NC + lax.axis_index("c")

    pltpu.sync_copy(w_hbm, w_v)
    pltpu.sync_copy(b_hbm, b_v)
    pltpu.sync_copy(offs_hbm, offs_v)

    zv = jnp.zeros((L,), jnp.float32)
    for s in range(SEG_PER_W):
        for k in range(D // L):
            acc_v[s, pl.ds(k * L, L)] = zv

    ws = [w_v[pl.ds(k * L, L)] for k in range(D // L)]
    b_bc = _dg(b_v[pl.ds(0, L)], lax.iota(jnp.int32, L) * 0)

    seg_lo = wid * SEG_PER_W
    ovec = offs_v[pl.ds(seg_lo, L)]
    start = ovec[0]
    end = ovec[SEG_PER_W]
    first = (start // 8) * 8
    nblk = lax.max((end - first + BLK - 1) // BLK, 0)

    def blk_body(i, carry):
        p = first + i * BLK
        bs = jnp.minimum(p, N - BLK)
        pltpu.sync_copy(x_hbm.at[pl.ds(bs, BLK)], xb_v)
        pltpu.sync_copy(batch_hbm.at[pl.ds(bs, BLK)], idx_v.at[pl.ds(0, BLK)])
        lo = lax.max(p, start)

        @plsc.parallel_loop(0, BLK, 1, unroll=4)
        def _rows(r):
            iv = idx_v[pl.ds(r, L)]
            seg = (iv[0] - seg_lo) & (SEG_PER_W - 1)
            g = bs + r
            pen = jnp.where((g >= lo) & (g < end), 0.0, -1e30)
            xs = [xb_v[r, pl.ds(k * L, L)] for k in range(D // L)]
            m = [xs[k] * ws[k] for k in range(D // L)]
            s01 = (m[0] + m[1]) + (m[2] + m[3])
            s23 = (m[4] + m[5]) + (m[6] + m[7])
            zv = _lane_allsum(s01 + s23) + b_bc + pen
            wt = 1.0 / (1.0 + jnp.exp(-zv))
            for k in range(D // L):
                plsc.addupdate(acc_v.at[seg, pl.ds(k * L, L)], xs[k] * wt)

        return carry

    lax.fori_loop(0, nblk, blk_body, 0, unroll=False)

    pltpu.sync_copy(acc_v, out_hbm.at[pl.ds(seg_lo, SEG_PER_W)])


@jax.jit
def kernel(x, batch, W, b):
    batch_i = batch.astype(jnp.int32)
    offs = jnp.searchsorted(batch_i, jnp.arange(NUM_SEGMENTS + 1, dtype=jnp.int32)
                            ).astype(jnp.int32)
    offs = jnp.pad(offs, (0, 7), constant_values=N)
    w_flat = W.reshape(D).astype(jnp.float32)
    b_pad = jnp.pad(b.astype(jnp.float32), (0, L - 1))

    mesh = plsc.VectorSubcoreMesh(core_axis_name="c", subcore_axis_name="s",
                                  num_cores=NC, num_subcores=NS)
    run = pl.kernel(
        _sc_body,
        out_type=jax.ShapeDtypeStruct((NUM_SEGMENTS, D), jnp.float32),
        mesh=mesh,
        scratch_types=[
            pltpu.VMEM((D,), jnp.float32),
            pltpu.VMEM((L,), jnp.float32),
            pltpu.VMEM((NUM_SEGMENTS + 8,), jnp.int32),
            pltpu.VMEM((BLK, D), jnp.float32),
            pltpu.VMEM((BLK + L,), jnp.int32),
            pltpu.VMEM((SEG_PER_W, D), jnp.float32),
        ],
    )
    return run(x, batch_i, w_flat, b_pad, offs)
